# baseline (speedup 1.0000x reference)
import functools

import jax
import jax.numpy as jnp
from jax import lax
from jax.experimental import pallas as pl
from jax.experimental.pallas import tpu as pltpu
from jax.experimental.pallas import tpu_sc as plsc

NC = 2
NS = 16
CHUNK = 128


IB = 8
CPW = 80


def _sc_scatter(n, r, interpret=False):
    rows_per_tile = n // NS

    mesh = plsc.VectorSubcoreMesh(core_axis_name="c", subcore_axis_name="s",
                                  num_cores=NC, num_subcores=NS)

    @functools.partial(
        pl.kernel,
        out_type=jax.ShapeDtypeStruct((NC, n, r), jnp.float32),
        mesh=mesh,
        scratch_types=[
            pltpu.VMEM((IB, CHUNK), jnp.int32),
            pltpu.VMEM((IB, CHUNK), jnp.int32),
            pltpu.VMEM((CHUNK,), jnp.int32),
            pltpu.VMEM((CHUNK,), jnp.int32),
            pltpu.VMEM((CHUNK, r), jnp.float32),
            pltpu.VMEM_SHARED((n, r), jnp.float32),
            pltpu.SemaphoreType.DMA,
        ],
        compiler_params=pltpu.CompilerParams(use_tc_tiling_on_sc=False),
        interpret=interpret,
    )
    def body(xaug_hbm, src_hbm, dst_hbm, zero_hbm, out_hbm,
             src_blk, dst_blk, src_v, dst_v, rows_v, acc_sh, sem):
        cid = lax.axis_index("c")
        sid = lax.axis_index("s")
        wid = sid * NC + cid

        r0 = sid * rows_per_tile
        pltpu.sync_copy(zero_hbm.at[pl.ds(r0, rows_per_tile)],
                        acc_sh.at[pl.ds(r0, rows_per_tile)])
        plsc.subcore_barrier()

        def group(g, _):
            c0 = wid * CPW + g * IB
            pltpu.sync_copy(src_hbm.at[pl.ds(c0, IB)], src_blk)
            pltpu.sync_copy(dst_hbm.at[pl.ds(c0, IB)], dst_blk)
            for k in range(IB):
                for i in range(CHUNK // 16):
                    sl = pl.ds(i * 16, 16)
                    src_v[sl] = src_blk[k, sl]
                    dst_v[sl] = dst_blk[k, sl]
                pltpu.async_copy(xaug_hbm.at[src_v], rows_v, sem).wait()
                pltpu.sync_copy(rows_v, acc_sh.at[dst_v], add=True)
            return None

        lax.fori_loop(0, CPW // IB, group, None)
        plsc.subcore_barrier()

        pltpu.sync_copy(acc_sh.at[pl.ds(r0, rows_per_tile)],
                        out_hbm.at[cid, pl.ds(r0, rows_per_tile)])

    return body


def _tc_mlp(n, d, r, interpret=False):
    bn = 2000
    assert n % bn == 0

    def body(acc_ref, noise_ref, w1_ref, b1_ref, w2_ref, b2_ref, out_ref):
        a = acc_ref[0]
        b = acc_ref[1]
        summed = a[:, :d] + b[:, :d]
        cnt = a[:, d:d + 1] + b[:, d:d + 1]
        g = summed / jnp.maximum(cnt, 1.0) + noise_ref[...]
        h = jnp.maximum(
            jnp.dot(g, w1_ref[...], preferred_element_type=jnp.float32)
            + b1_ref[...], 0.0)
        o = jnp.maximum(
            jnp.dot(h, w2_ref[...], preferred_element_type=jnp.float32)
            + b2_ref[...], 0.0)
        out_ref[...] = o

    dh = d // 2
    return pl.pallas_call(
        body,
        grid=(n // bn,),
        in_specs=[
            pl.BlockSpec((NC, bn, r), lambda i: (0, i, 0)),
            pl.BlockSpec((bn, d), lambda i: (i, 0)),
            pl.BlockSpec((d, dh), lambda i: (0, 0)),
            pl.BlockSpec((1, dh), lambda i: (0, 0)),
            pl.BlockSpec((dh, d), lambda i: (0, 0)),
            pl.BlockSpec((1, d), lambda i: (0, 0)),
        ],
        out_specs=pl.BlockSpec((bn, d), lambda i: (i, 0)),
        out_shape=jax.ShapeDtypeStruct((n, d), jnp.float32),
        interpret=interpret,
    )


def kernel(x, edge_index, batch, W1, b1, W2, b2, noise):
    n, d = x.shape
    e = edge_index.shape[1]
    r = 144

    ones_pad = jnp.concatenate(
        [jnp.ones((n, 1), jnp.float32), jnp.zeros((n, r - d - 1), jnp.float32)],
        axis=1)
    x_aug = jnp.concatenate([x, ones_pad], axis=1)
    x_aug = jnp.concatenate([x_aug, jnp.zeros((8, r), jnp.float32)], axis=0)

    e_pad = NC * NS * CPW * CHUNK
    src = jnp.concatenate(
        [edge_index[0], jnp.full((e_pad - e,), n, jnp.int32)]).reshape(-1, CHUNK)
    dst = jnp.concatenate(
        [edge_index[1],
         jnp.arange(e_pad - e, dtype=jnp.int32) % n]).reshape(-1, CHUNK)

    acc = _sc_scatter(n, r)(x_aug, src, dst, jnp.zeros((n, r), jnp.float32))
    return _tc_mlp(n, d, r)(acc, noise, W1, b1.reshape(1, -1), W2,
                            b2.reshape(1, -1))


# speedup vs baseline: 1.7718x; 1.7718x over previous
import functools

import jax
import jax.numpy as jnp
from jax import lax
from jax.experimental import pallas as pl
from jax.experimental.pallas import tpu as pltpu
from jax.experimental.pallas import tpu_sc as plsc

NC = 2
NS = 16
CHUNK = 128


def _sc_scatter(n, e, r, interpret=False):
    num_chunks = e // CHUNK
    nw = NC * NS
    cpw = -(-num_chunks // nw)
    rows_per_tile = n // NS

    mesh = plsc.VectorSubcoreMesh(core_axis_name="c", subcore_axis_name="s",
                                  num_cores=NC, num_subcores=NS)

    @functools.partial(
        pl.kernel,
        out_type=jax.ShapeDtypeStruct((NC, n, r), jnp.float32),
        mesh=mesh,
        scratch_types=[
            pltpu.VMEM((CHUNK,), jnp.int32),
            pltpu.VMEM((CHUNK,), jnp.int32),
            pltpu.VMEM((CHUNK, r), jnp.float32),
            pltpu.VMEM_SHARED((n, r), jnp.float32),
            pltpu.SemaphoreType.DMA,
        ],
        compiler_params=pltpu.CompilerParams(use_tc_tiling_on_sc=False),
        interpret=interpret,
    )
    def body(xaug_hbm, src_hbm, dst_hbm, zero_hbm, out_hbm,
             src_v, dst_v, rows_v, acc_sh, sem):
        cid = lax.axis_index("c")
        sid = lax.axis_index("s")
        wid = sid * NC + cid

        r0 = sid * rows_per_tile
        pltpu.sync_copy(zero_hbm.at[pl.ds(r0, rows_per_tile)],
                        acc_sh.at[pl.ds(r0, rows_per_tile)])
        plsc.subcore_barrier()

        def step(j, _):
            chunk = wid * cpw + j

            @pl.when(chunk < num_chunks)
            def _():
                base = chunk * CHUNK
                pltpu.sync_copy(src_hbm.at[pl.ds(base, CHUNK)], src_v)
                pltpu.sync_copy(dst_hbm.at[pl.ds(base, CHUNK)], dst_v)
                pltpu.async_copy(xaug_hbm.at[src_v], rows_v, sem).wait()
                pltpu.sync_copy(rows_v, acc_sh.at[dst_v], add=True)

            return _

        lax.fori_loop(0, cpw, step, None)
        plsc.subcore_barrier()

        pltpu.sync_copy(acc_sh.at[pl.ds(r0, rows_per_tile)],
                        out_hbm.at[cid, pl.ds(r0, rows_per_tile)])

    return body


def _tc_mlp(n, d, r, interpret=False):
    bn = 2000
    assert n % bn == 0

    def body(acc_ref, noise_ref, w1_ref, b1_ref, w2_ref, b2_ref, out_ref):
        a = acc_ref[0]
        b = acc_ref[1]
        summed = a[:, :d] + b[:, :d]
        cnt = a[:, d:d + 1] + b[:, d:d + 1]
        g = summed / jnp.maximum(cnt, 1.0) + noise_ref[...]
        h = jnp.maximum(
            jnp.dot(g, w1_ref[...], preferred_element_type=jnp.float32)
            + b1_ref[...], 0.0)
        o = jnp.maximum(
            jnp.dot(h, w2_ref[...], preferred_element_type=jnp.float32)
            + b2_ref[...], 0.0)
        out_ref[...] = o

    dh = d // 2
    return pl.pallas_call(
        body,
        grid=(n // bn,),
        in_specs=[
            pl.BlockSpec((NC, bn, r), lambda i: (0, i, 0)),
            pl.BlockSpec((bn, d), lambda i: (i, 0)),
            pl.BlockSpec((d, dh), lambda i: (0, 0)),
            pl.BlockSpec((1, dh), lambda i: (0, 0)),
            pl.BlockSpec((dh, d), lambda i: (0, 0)),
            pl.BlockSpec((1, d), lambda i: (0, 0)),
        ],
        out_specs=pl.BlockSpec((bn, d), lambda i: (i, 0)),
        out_shape=jax.ShapeDtypeStruct((n, d), jnp.float32),
        interpret=interpret,
    )


def kernel(x, edge_index, batch, W1, b1, W2, b2, noise):
    n, d = x.shape
    e = edge_index.shape[1]
    r = 144

    ones_pad = jnp.concatenate(
        [jnp.ones((n, 1), jnp.float32), jnp.zeros((n, r - d - 1), jnp.float32)],
        axis=1)
    x_aug = jnp.concatenate([x, ones_pad], axis=1)

    acc = _sc_scatter(n, e, r)(x_aug, edge_index[0], edge_index[1],
                               jnp.zeros((n, r), jnp.float32))
    return _tc_mlp(n, d, r)(acc, noise, W1, b1.reshape(1, -1), W2,
                            b2.reshape(1, -1))
